# baseline (speedup 1.0000x reference)
import functools

import jax
import jax.numpy as jnp
from jax import lax
from jax.experimental import pallas as pl
from jax.experimental.pallas import tpu as pltpu
from jax.experimental.pallas import tpu_sc as plsc

N_NODES = 10000
N_EDGES = 320000
D_FEAT = 128
N_CLASSES = 10

NC = 2
NS = 16
NW = NC * NS
CHUNK = 80
EPW = N_EDGES // NW
NCHUNK = EPW // CHUNK
IBLK = 25
NBLK = NCHUNK // IBLK
N_ACC = N_NODES
NPAIR = D_FEAT // 2
DUMP_ROWS = 624
DUMP_LAST = N_NODES - 15 * DUMP_ROWS


def _sc_aggregate(x, xi, srcs, dsts):
    mesh = plsc.VectorSubcoreMesh(core_axis_name="c", subcore_axis_name="s")

    @functools.partial(
        pl.kernel,
        out_type=jax.ShapeDtypeStruct((NC, N_NODES, D_FEAT), jnp.float32),
        mesh=mesh,
        scratch_types=[
            pltpu.VMEM((EPW,), jnp.int32),
            pltpu.VMEM((IBLK, 1, CHUNK), jnp.int32),
            pltpu.VMEM((CHUNK, NPAIR), jnp.int32),
            pltpu.VMEM((CHUNK, NPAIR), jnp.int32),
            pltpu.VMEM((CHUNK, D_FEAT), jnp.float32),
            pltpu.VMEM((CHUNK, D_FEAT), jnp.float32),
            pltpu.VMEM_SHARED((N_ACC, D_FEAT), jnp.float32),
            pltpu.SemaphoreType.DMA,
            pltpu.SemaphoreType.DMA,
            pltpu.SemaphoreType.DMA,
            pltpu.SemaphoreType.DMA,
        ],
        compiler_params=pltpu.CompilerParams(use_tc_tiling_on_sc=False),
    )
    def sc_agg(x_hbm, xi_hbm, src_hbm, dst_hbm, out_hbm, src_v, dst_v,
               bf_a, bf_b, f_a, f_b, acc_sh, g_a, g_b, s_a, s_b):
        cid = lax.axis_index("c")
        sid = lax.axis_index("s")
        wid = cid * NS + sid

        @pl.when(sid < 15)
        def _():
            pltpu.sync_copy(x_hbm.at[pl.ds(sid * DUMP_ROWS, DUMP_ROWS)],
                            acc_sh.at[pl.ds(sid * DUMP_ROWS, DUMP_ROWS)])

        @pl.when(sid == 15)
        def _():
            pltpu.sync_copy(x_hbm.at[pl.ds(15 * DUMP_ROWS, DUMP_LAST)],
                            acc_sh.at[pl.ds(15 * DUMP_ROWS, DUMP_LAST)])

        pltpu.sync_copy(src_hbm.at[wid], src_v)
        plsc.subcore_barrier()

        def src_idx(gj):
            return src_v.at[pl.ds(gj * CHUNK, CHUNK)]

        def unpack_rows(bf, f):
            shift16 = jnp.full((16,), 16, jnp.int32)
            maskhi = jnp.full((16,), -65536, jnp.int32)

            def conv_row(r, carry):
                for c in range(4):
                    v = bf[r, pl.ds(16 * c, 16)]
                    lo = lax.bitcast_convert_type(v << shift16, jnp.float32)
                    hi = lax.bitcast_convert_type(v & maskhi, jnp.float32)
                    f[r, pl.ds(32 * c, 16)] = lo
                    f[r, pl.ds(32 * c + 16, 16)] = hi
                return carry

            lax.fori_loop(0, CHUNK, conv_row, 0)

        def outer(b, carry):
            base = b * IBLK
            pltpu.sync_copy(dst_hbm.at[wid, b], dst_v)
            pltpu.async_copy(xi_hbm.at[src_idx(base)], bf_a, g_a)
            pltpu.async_copy(xi_hbm.at[src_idx(base + 1)], bf_b, g_b)

            def body(i, carry):
                j = 2 * i
                gj = base + j

                pltpu.make_async_copy(xi_hbm.at[src_idx(gj)], bf_a,
                                      g_a).wait()

                @pl.when(i > 0)
                def _():
                    pltpu.make_async_copy(f_a, acc_sh.at[dst_v.at[0, 0]],
                                          s_a).wait()

                unpack_rows(bf_a, f_a)

                @pl.when(j + 2 < IBLK)
                def _():
                    pltpu.async_copy(xi_hbm.at[src_idx(gj + 2)], bf_a, g_a)

                pltpu.async_copy(f_a, acc_sh.at[dst_v.at[j, 0]], s_a,
                                 add=True)

                @pl.when(j + 1 < IBLK)
                def _():
                    pltpu.make_async_copy(xi_hbm.at[src_idx(gj + 1)], bf_b,
                                          g_b).wait()

                    @pl.when(i > 0)
                    def _():
                        pltpu.make_async_copy(f_b, acc_sh.at[dst_v.at[0, 0]],
                                              s_b).wait()

                    unpack_rows(bf_b, f_b)

                    @pl.when(j + 3 < IBLK)
                    def _():
                        pltpu.async_copy(xi_hbm.at[src_idx(gj + 3)], bf_b,
                                         g_b)

                    pltpu.async_copy(f_b, acc_sh.at[dst_v.at[j + 1, 0]], s_b,
                                     add=True)

                return carry

            lax.fori_loop(0, (IBLK + 1) // 2, body, 0)
            pltpu.make_async_copy(f_a, acc_sh.at[dst_v.at[0, 0]], s_a).wait()
            pltpu.make_async_copy(f_b, acc_sh.at[dst_v.at[0, 0]], s_b).wait()
            return carry

        lax.fori_loop(0, NBLK, outer, 0)
        plsc.subcore_barrier()

        @pl.when(sid < 15)
        def _():
            pltpu.sync_copy(
                acc_sh.at[pl.ds(sid * DUMP_ROWS, DUMP_ROWS)],
                out_hbm.at[cid, pl.ds(sid * DUMP_ROWS, DUMP_ROWS)],
            )

        @pl.when(sid == 15)
        def _():
            pltpu.sync_copy(
                acc_sh.at[pl.ds(15 * DUMP_ROWS, DUMP_LAST)],
                out_hbm.at[cid, pl.ds(15 * DUMP_ROWS, DUMP_LAST)],
            )

    return sc_agg(x, xi, srcs, dsts)


def _tc_mlp(p0, p1, x, W1, b1, W2, b2):
    BN = 1000
    grid = (N_NODES // BN,)

    def body(p0_ref, p1_ref, x_ref, w1_ref, b1_ref, w2_ref, b2_ref, out_ref):
        s = p0_ref[...] + p1_ref[...] - x_ref[...]
        h = jnp.dot(s, w1_ref[...], preferred_element_type=jnp.float32)
        h = jnp.maximum(h + b1_ref[...], 0.0)
        out_ref[...] = (
            jnp.dot(h, w2_ref[...], preferred_element_type=jnp.float32)
            + b2_ref[...]
        )

    row_spec = pl.BlockSpec((BN, D_FEAT), lambda i: (i, 0))
    full = lambda shape: pl.BlockSpec(shape, lambda i: (0,) * len(shape))
    return pl.pallas_call(
        body,
        grid=grid,
        in_specs=[
            row_spec, row_spec, row_spec,
            full((D_FEAT, D_FEAT)), full((1, D_FEAT)),
            full((D_FEAT, N_CLASSES)), full((1, N_CLASSES)),
        ],
        out_specs=pl.BlockSpec((BN, N_CLASSES), lambda i: (i, 0)),
        out_shape=jax.ShapeDtypeStruct((N_NODES, N_CLASSES), jnp.float32),
    )(p0, p1, x, W1, b1, W2, b2)


def kernel(x, edge_index, W1, b1, W2, b2):
    xb = x.astype(jnp.bfloat16).reshape(N_NODES, 4, 2, 16)
    xb = jnp.transpose(xb, (0, 1, 3, 2))
    xi = jax.lax.bitcast_convert_type(xb, jnp.int32).reshape(N_NODES, NPAIR)

    srcs = edge_index[0].reshape(NW, EPW)
    dsts = edge_index[1].reshape(NW, NBLK, IBLK, 1, CHUNK)
    partials = _sc_aggregate(x, xi, srcs, dsts)
    return _tc_mlp(partials[0], partials[1], x, W1, b1.reshape(1, D_FEAT),
                   W2, b2.reshape(1, N_CLASSES))


# speedup vs baseline: 1.5762x; 1.5762x over previous
import functools

import jax
import jax.numpy as jnp
from jax import lax
from jax.experimental import pallas as pl
from jax.experimental.pallas import tpu as pltpu
from jax.experimental.pallas import tpu_sc as plsc

N_NODES = 10000
N_EDGES = 320000
D_FEAT = 128
N_CLASSES = 10

NC = 2
NS = 16
NW = NC * NS
CHUNK = 80
EPW = N_EDGES // NW
NCHUNK = EPW // CHUNK
IBLK = 25
NBLK = NCHUNK // IBLK
N_ACC = N_NODES
NPAIR = D_FEAT // 2
DUMP_ROWS = 624
DUMP_LAST = N_NODES - 15 * DUMP_ROWS


def _sc_aggregate(x, xi, srcs, dsts):
    mesh = plsc.VectorSubcoreMesh(core_axis_name="c", subcore_axis_name="s")

    @functools.partial(
        pl.kernel,
        out_type=jax.ShapeDtypeStruct((NC, N_NODES, D_FEAT), jnp.float32),
        mesh=mesh,
        scratch_types=[
            pltpu.VMEM((EPW,), jnp.int32),
            pltpu.VMEM((IBLK, 1, CHUNK), jnp.int32),
            pltpu.VMEM((CHUNK, NPAIR), jnp.int32),
            pltpu.VMEM((CHUNK, NPAIR), jnp.int32),
            pltpu.VMEM((CHUNK, D_FEAT), jnp.float32),
            pltpu.VMEM((CHUNK, D_FEAT), jnp.float32),
            pltpu.VMEM_SHARED((N_ACC, D_FEAT), jnp.float32),
            pltpu.SemaphoreType.DMA,
            pltpu.SemaphoreType.DMA,
            pltpu.SemaphoreType.DMA,
            pltpu.SemaphoreType.DMA,
        ],
        compiler_params=pltpu.CompilerParams(use_tc_tiling_on_sc=False),
    )
    def sc_agg(x_hbm, xi_hbm, src_hbm, dst_hbm, out_hbm, src_v, dst_v,
               bf_a, bf_b, f_a, f_b, acc_sh, g_a, g_b, s_a, s_b):
        cid = lax.axis_index("c")
        sid = lax.axis_index("s")
        wid = cid * NS + sid

        @pl.when(sid < 15)
        def _():
            pltpu.sync_copy(x_hbm.at[pl.ds(sid * DUMP_ROWS, DUMP_ROWS)],
                            acc_sh.at[pl.ds(sid * DUMP_ROWS, DUMP_ROWS)])

        @pl.when(sid == 15)
        def _():
            pltpu.sync_copy(x_hbm.at[pl.ds(15 * DUMP_ROWS, DUMP_LAST)],
                            acc_sh.at[pl.ds(15 * DUMP_ROWS, DUMP_LAST)])

        pltpu.sync_copy(src_hbm.at[wid], src_v)
        plsc.subcore_barrier()

        def src_idx(gj):
            return src_v.at[pl.ds(gj * CHUNK, CHUNK)]

        def unpack_rows(bf, f):
            shift16 = jnp.full((16,), 16, jnp.int32)
            maskhi = jnp.full((16,), -65536, jnp.int32)

            @plsc.parallel_loop(0, CHUNK, step=1, unroll=8)
            def conv_row(r):
                for c in range(4):
                    v = bf[r, pl.ds(16 * c, 16)]
                    lo = lax.bitcast_convert_type(v << shift16, jnp.float32)
                    hi = lax.bitcast_convert_type(v & maskhi, jnp.float32)
                    f[r, pl.ds(32 * c, 16)] = lo
                    f[r, pl.ds(32 * c + 16, 16)] = hi

        def outer(b, carry):
            base = b * IBLK
            pltpu.sync_copy(dst_hbm.at[wid, b], dst_v)
            pltpu.async_copy(xi_hbm.at[src_idx(base)], bf_a, g_a)
            pltpu.async_copy(xi_hbm.at[src_idx(base + 1)], bf_b, g_b)

            def body(i, carry):
                j = 2 * i
                gj = base + j

                pltpu.make_async_copy(xi_hbm.at[src_idx(gj)], bf_a,
                                      g_a).wait()

                @pl.when(i > 0)
                def _():
                    pltpu.make_async_copy(f_a, acc_sh.at[dst_v.at[0, 0]],
                                          s_a).wait()

                unpack_rows(bf_a, f_a)

                @pl.when(j + 2 < IBLK)
                def _():
                    pltpu.async_copy(xi_hbm.at[src_idx(gj + 2)], bf_a, g_a)

                pltpu.async_copy(f_a, acc_sh.at[dst_v.at[j, 0]], s_a,
                                 add=True)

                @pl.when(j + 1 < IBLK)
                def _():
                    pltpu.make_async_copy(xi_hbm.at[src_idx(gj + 1)], bf_b,
                                          g_b).wait()

                    @pl.when(i > 0)
                    def _():
                        pltpu.make_async_copy(f_b, acc_sh.at[dst_v.at[0, 0]],
                                              s_b).wait()

                    unpack_rows(bf_b, f_b)

                    @pl.when(j + 3 < IBLK)
                    def _():
                        pltpu.async_copy(xi_hbm.at[src_idx(gj + 3)], bf_b,
                                         g_b)

                    pltpu.async_copy(f_b, acc_sh.at[dst_v.at[j + 1, 0]], s_b,
                                     add=True)

                return carry

            lax.fori_loop(0, (IBLK + 1) // 2, body, 0)
            pltpu.make_async_copy(f_a, acc_sh.at[dst_v.at[0, 0]], s_a).wait()
            pltpu.make_async_copy(f_b, acc_sh.at[dst_v.at[0, 0]], s_b).wait()
            return carry

        lax.fori_loop(0, NBLK, outer, 0)
        plsc.subcore_barrier()

        @pl.when(sid < 15)
        def _():
            pltpu.sync_copy(
                acc_sh.at[pl.ds(sid * DUMP_ROWS, DUMP_ROWS)],
                out_hbm.at[cid, pl.ds(sid * DUMP_ROWS, DUMP_ROWS)],
            )

        @pl.when(sid == 15)
        def _():
            pltpu.sync_copy(
                acc_sh.at[pl.ds(15 * DUMP_ROWS, DUMP_LAST)],
                out_hbm.at[cid, pl.ds(15 * DUMP_ROWS, DUMP_LAST)],
            )

    return sc_agg(x, xi, srcs, dsts)


def _tc_mlp(p0, p1, x, W1, b1, W2, b2):
    BN = 1000
    grid = (N_NODES // BN,)

    def body(p0_ref, p1_ref, x_ref, w1_ref, b1_ref, w2_ref, b2_ref, out_ref):
        s = p0_ref[...] + p1_ref[...] - x_ref[...]
        h = jnp.dot(s, w1_ref[...], preferred_element_type=jnp.float32)
        h = jnp.maximum(h + b1_ref[...], 0.0)
        out_ref[...] = (
            jnp.dot(h, w2_ref[...], preferred_element_type=jnp.float32)
            + b2_ref[...]
        )

    row_spec = pl.BlockSpec((BN, D_FEAT), lambda i: (i, 0))
    full = lambda shape: pl.BlockSpec(shape, lambda i: (0,) * len(shape))
    return pl.pallas_call(
        body,
        grid=grid,
        in_specs=[
            row_spec, row_spec, row_spec,
            full((D_FEAT, D_FEAT)), full((1, D_FEAT)),
            full((D_FEAT, N_CLASSES)), full((1, N_CLASSES)),
        ],
        out_specs=pl.BlockSpec((BN, N_CLASSES), lambda i: (i, 0)),
        out_shape=jax.ShapeDtypeStruct((N_NODES, N_CLASSES), jnp.float32),
    )(p0, p1, x, W1, b1, W2, b2)


def kernel(x, edge_index, W1, b1, W2, b2):
    xb = x.astype(jnp.bfloat16).reshape(N_NODES, 4, 2, 16)
    xb = jnp.transpose(xb, (0, 1, 3, 2))
    xi = jax.lax.bitcast_convert_type(xb, jnp.int32).reshape(N_NODES, NPAIR)

    srcs = edge_index[0].reshape(NW, EPW)
    dsts = edge_index[1].reshape(NW, NBLK, IBLK, 1, CHUNK)
    partials = _sc_aggregate(x, xi, srcs, dsts)
    return _tc_mlp(partials[0], partials[1], x, W1, b1.reshape(1, D_FEAT),
                   W2, b2.reshape(1, N_CLASSES))
